# Initial kernel scaffold; baseline (speedup 1.0000x reference)
import functools

import jax
import jax.numpy as jnp
from jax import lax
from jax.experimental import pallas as pl
from jax.experimental.pallas import tpu as pltpu
from jax.experimental.pallas import tpu_sc as plsc

NC = 2
NS = 16
NW = NC * NS
EPS = 1e-5
BD = 80
BM = 80


def _rsqrt16(x):
    i = plsc.bitcast(x, jnp.int32)
    y = plsc.bitcast(0x5F3759DF - (i >> 1), jnp.float32)
    for _ in range(4):
        y = y * (1.5 - 0.5 * x * y * y)
    return y


def _matmul_body(x_ref, w_ref, o_ref):
    o_ref[...] = jnp.dot(x_ref[...], w_ref[...],
                         preferred_element_type=jnp.float32)


def _matmul(x, w):
    n, _ = x.shape
    m = w.shape[1]
    return pl.pallas_call(
        _matmul_body,
        out_shape=jax.ShapeDtypeStruct((n, m), jnp.float32),
    )(x, w)


def _combine_body(with_mm, p_ref, h_ref, dinv_ref, b_ref, g_ref, be_ref,
                  w_ref, o_ref):
    dinv = dinv_ref[...]
    a = p_ref[0] + p_ref[1] + h_ref[...] * (dinv * dinv) + b_ref[...]
    mean = jnp.mean(a, axis=0, keepdims=True)
    var = jnp.mean((a - mean) * (a - mean), axis=0, keepdims=True)
    xh = (a - mean) * lax.rsqrt(var + EPS)
    y = jnp.maximum(xh * g_ref[...] + be_ref[...], 0.0)
    if with_mm:
        y = jnp.dot(y, w_ref[...], preferred_element_type=jnp.float32)
    o_ref[...] = y


def _combine(p, h, dinv2d, b, g, be, w, with_mm):
    n, d = h.shape
    m = w.shape[1] if with_mm else d
    return pl.pallas_call(
        functools.partial(_combine_body, with_mm),
        out_shape=jax.ShapeDtypeStruct((n, m), jnp.float32),
    )(p, h, dinv2d, b.reshape(1, d), g.reshape(1, d), be.reshape(1, d), w)


def _sc_body(n, e, d, compute_deg,
             row_h, col_h, ew_h, h_h, dinv_in_h, znd_h, zn_h,
             part_h, dinv_out_h,
             meta_r, meta_c, meta_w, dinv_v, rows_v,
             deg_s, out_s, sem):
    c = lax.axis_index("c")
    s = lax.axis_index("s")
    wid = c * NS + s
    epw = e // NW

    @pl.when(s == 0)
    def _():
        pltpu.sync_copy(znd_h, out_s)
        if compute_deg:
            pltpu.sync_copy(zn_h, deg_s)
    plsc.subcore_barrier()

    if compute_deg:
        eps_sc = e // NS

        def deg_batch(i, carry):
            base = s * eps_sc + i * BD
            pltpu.sync_copy(col_h.at[pl.ds(base, BD)], meta_c)
            pltpu.sync_copy(ew_h.at[pl.ds(base, BD)], meta_w)
            pltpu.sync_copy(meta_w, deg_s.at[meta_c], add=True)
            return carry

        lax.fori_loop(0, eps_sc // BD, deg_batch, 0)
        plsc.subcore_barrier()

        pltpu.sync_copy(deg_s, dinv_v)

        def rs_chunk(i, carry):
            sl = pl.ds(i * 16, 16)
            dinv_v[sl] = _rsqrt16(dinv_v[sl] + 1.0)
            return carry

        lax.fori_loop(0, n // 16, rs_chunk, 0)

        @pl.when(jnp.logical_and(c == 0, s == 0))
        def _():
            pltpu.sync_copy(dinv_v, dinv_out_h)
    else:
        pltpu.sync_copy(dinv_in_h, dinv_v)

    def msg_batch(i, carry):
        base = wid * epw + i * BM
        pltpu.sync_copy(row_h.at[pl.ds(base, BM)], meta_r)
        pltpu.sync_copy(col_h.at[pl.ds(base, BM)], meta_c)
        pltpu.sync_copy(ew_h.at[pl.ds(base, BM)], meta_w)
        pltpu.async_copy(h_h.at[meta_r], rows_v, sem).wait()

        def norm16(j, cin):
            sl = pl.ds(j * 16, 16)
            dr = plsc.load_gather(dinv_v, [meta_r[sl]])
            dc = plsc.load_gather(dinv_v, [meta_c[sl]])
            meta_w[sl] = dr * meta_w[sl] * dc
            return cin

        lax.fori_loop(0, BM // 16, norm16, 0)

        def scale_edge(ei, cin):
            nrm = meta_w[ei]
            for f in range(d // 16):
                sl = (ei, pl.ds(f * 16, 16))
                rows_v[sl] = rows_v[sl] * nrm
            return cin

        lax.fori_loop(0, BM, scale_edge, 0)
        pltpu.sync_copy(rows_v, out_s.at[meta_c], add=True)
        return carry

    lax.fori_loop(0, epw // BM, msg_batch, 0)
    plsc.subcore_barrier()

    @pl.when(s == 0)
    def _():
        pltpu.sync_copy(out_s, part_h.at[c])


def _sc_pass(row, col, ew, h, dinv, compute_deg):
    n, d = h.shape
    e = row.shape[0]
    mesh = plsc.VectorSubcoreMesh(core_axis_name="c", subcore_axis_name="s")
    znd = jnp.zeros((n, d), jnp.float32)
    zn = jnp.zeros((n,), jnp.float32)
    if dinv is None:
        dinv = zn
    out_type = (jax.ShapeDtypeStruct((NC, n, d), jnp.float32),
                jax.ShapeDtypeStruct((n,), jnp.float32))
    scratch = [
        pltpu.VMEM((BM,), jnp.int32),
        pltpu.VMEM((BM,), jnp.int32),
        pltpu.VMEM((BM,), jnp.float32),
        pltpu.VMEM((n,), jnp.float32),
        pltpu.VMEM((BM, d), jnp.float32),
        pltpu.VMEM_SHARED((n,), jnp.float32),
        pltpu.VMEM_SHARED((n, d), jnp.float32),
        pltpu.SemaphoreType.DMA,
    ]
    fn = pl.kernel(
        functools.partial(_sc_body, n, e, d, compute_deg),
        out_type=out_type,
        mesh=mesh,
        scratch_types=scratch,
    )
    return fn(row, col, ew, h, dinv, znd, zn)


def kernel(x, edge_index, edge_weight, W1, b1, g1, be1, W2, b2, g2, be2):
    row = edge_index[0].astype(jnp.int32)
    col = edge_index[1].astype(jnp.int32)
    ew = edge_weight.astype(jnp.float32)

    h1 = _matmul(x, W1)
    p1, dinv = _sc_pass(row, col, ew, h1, None, True)
    dinv2d = dinv.reshape(-1, 1)
    h2 = _combine(p1, h1, dinv2d, b1, g1, be1, W2, True)
    p2, _ = _sc_pass(row, col, ew, h2, dinv, False)
    out = _combine(p2, h2, dinv2d, b2, g2, be2, W2, False)
    return out


# speedup vs baseline: 8.6537x; 8.6537x over previous
import functools

import jax
import jax.numpy as jnp
from jax import lax
from jax.experimental import pallas as pl
from jax.experimental.pallas import tpu as pltpu
from jax.experimental.pallas import tpu_sc as plsc

NC = 2
NS = 16
NW = NC * NS
EPS = 1e-5
BD = 80
BM = 80


def _rsqrt16(x):
    i = lax.bitcast_convert_type(x, jnp.int32)
    y = lax.bitcast_convert_type(0x5F3759DF - (i >> 1), jnp.float32)
    for _ in range(4):
        y = y * (1.5 - 0.5 * x * y * y)
    return y


def _matmul_body(x_ref, w_ref, o_ref):
    o_ref[...] = jnp.dot(x_ref[...], w_ref[...],
                         preferred_element_type=jnp.float32)


def _matmul(x, w):
    n, _ = x.shape
    m = w.shape[1]
    return pl.pallas_call(
        _matmul_body,
        out_shape=jax.ShapeDtypeStruct((n, m), jnp.float32),
    )(x, w)


def _combine_body(with_mm, p_ref, h_ref, dinv_ref, b_ref, g_ref, be_ref,
                  w_ref, o_ref):
    dinv = dinv_ref[...]
    a = p_ref[0] + p_ref[1] + h_ref[...] * (dinv * dinv) + b_ref[...]
    mean = jnp.mean(a, axis=0, keepdims=True)
    var = jnp.mean((a - mean) * (a - mean), axis=0, keepdims=True)
    xh = (a - mean) * lax.rsqrt(var + EPS)
    y = jnp.maximum(xh * g_ref[...] + be_ref[...], 0.0)
    if with_mm:
        y = jnp.dot(y, w_ref[...], preferred_element_type=jnp.float32)
    o_ref[...] = y


def _combine(p, h, dinv2d, b, g, be, w, with_mm):
    n, d = h.shape
    m = w.shape[1] if with_mm else d
    return pl.pallas_call(
        functools.partial(_combine_body, with_mm),
        out_shape=jax.ShapeDtypeStruct((n, m), jnp.float32),
    )(p, h, dinv2d, b.reshape(1, d), g.reshape(1, d), be.reshape(1, d), w)


def _sc_body(n, e, d, compute_deg,
             row_h, col_h, ew_h, h_h, dinv_in_h, znd_h, zn_h,
             part_h, dinv_out_h,
             meta_r, meta_c, meta_w, dinv_v, rows_v,
             deg_s, out_s, sem):
    c = lax.axis_index("c")
    s = lax.axis_index("s")
    wid = c * NS + s
    epw = e // NW

    @pl.when(s == 0)
    def _():
        pltpu.sync_copy(znd_h, out_s)
        if compute_deg:
            pltpu.sync_copy(zn_h, deg_s)
    plsc.subcore_barrier()

    if compute_deg:
        eps_sc = e // NS

        def deg_batch(i, carry):
            base = s * eps_sc + i * BD
            pltpu.sync_copy(col_h.at[pl.ds(base, BD)], meta_c)
            pltpu.sync_copy(ew_h.at[pl.ds(base, BD)], meta_w)
            pltpu.sync_copy(meta_w, deg_s.at[meta_c], add=True)
            return carry

        lax.fori_loop(0, eps_sc // BD, deg_batch, 0)
        plsc.subcore_barrier()

        pltpu.sync_copy(deg_s, dinv_v)

        def rs_chunk(i, carry):
            sl = pl.ds(i * 16, 16)
            dinv_v[sl] = _rsqrt16(dinv_v[sl] + 1.0)
            return carry

        lax.fori_loop(0, n // 16, rs_chunk, 0)

        @pl.when(jnp.logical_and(c == 0, s == 0))
        def _():
            pltpu.sync_copy(dinv_v, dinv_out_h)
    else:
        pltpu.sync_copy(dinv_in_h, dinv_v)

    def msg_batch(i, carry):
        base = wid * epw + i * BM
        pltpu.sync_copy(row_h.at[pl.ds(base, BM)], meta_r)
        pltpu.sync_copy(col_h.at[pl.ds(base, BM)], meta_c)
        pltpu.sync_copy(ew_h.at[pl.ds(base, BM)], meta_w)
        pltpu.async_copy(h_h.at[meta_r], rows_v, sem).wait()

        def norm16(j, cin):
            sl = pl.ds(j * 16, 16)
            dr = plsc.load_gather(dinv_v, [meta_r[sl]])
            dc = plsc.load_gather(dinv_v, [meta_c[sl]])
            meta_w[sl] = dr * meta_w[sl] * dc
            return cin

        lax.fori_loop(0, BM // 16, norm16, 0)

        def scale16(j, cin):
            nv = meta_w[pl.ds(j * 16, 16)]
            for k in range(16):
                nrm = nv[k]
                for f in range(d // 16):
                    sl = (j * 16 + k, pl.ds(f * 16, 16))
                    rows_v[sl] = rows_v[sl] * nrm
            return cin

        lax.fori_loop(0, BM // 16, scale16, 0)
        pltpu.sync_copy(rows_v, out_s.at[meta_c], add=True)
        return carry

    lax.fori_loop(0, epw // BM, msg_batch, 0)
    plsc.subcore_barrier()

    @pl.when(s == 0)
    def _():
        pltpu.sync_copy(out_s, part_h.at[c])


def _sc_pass(row, col, ew, h, dinv, compute_deg):
    n, d = h.shape
    e = row.shape[0]
    mesh = plsc.VectorSubcoreMesh(core_axis_name="c", subcore_axis_name="s")
    znd = jnp.zeros((n, d), jnp.float32)
    zn = jnp.zeros((n,), jnp.float32)
    if dinv is None:
        dinv = zn
    out_type = (jax.ShapeDtypeStruct((NC, n, d), jnp.float32),
                jax.ShapeDtypeStruct((n,), jnp.float32))
    scratch = [
        pltpu.VMEM((BM,), jnp.int32),
        pltpu.VMEM((BM,), jnp.int32),
        pltpu.VMEM((BM,), jnp.float32),
        pltpu.VMEM((n,), jnp.float32),
        pltpu.VMEM((BM, d), jnp.float32),
        pltpu.VMEM_SHARED((n,), jnp.float32),
        pltpu.VMEM_SHARED((n, d), jnp.float32),
        pltpu.SemaphoreType.DMA,
    ]
    fn = pl.kernel(
        functools.partial(_sc_body, n, e, d, compute_deg),
        out_type=out_type,
        mesh=mesh,
        scratch_types=scratch,
        compiler_params=pltpu.CompilerParams(needs_layout_passes=False),
    )
    return fn(row, col, ew, h, dinv, znd, zn)


def kernel(x, edge_index, edge_weight, W1, b1, g1, be1, W2, b2, g2, be2):
    row = edge_index[0].astype(jnp.int32)
    col = edge_index[1].astype(jnp.int32)
    ew = edge_weight.astype(jnp.float32)

    h1 = _matmul(x, W1)
    p1, dinv = _sc_pass(row, col, ew, h1, None, True)
    dinv2d = dinv.reshape(-1, 1)
    h2 = _combine(p1, h1, dinv2d, b1, g1, be1, W2, True)
    p2, _ = _sc_pass(row, col, ew, h2, dinv, False)
    out = _combine(p2, h2, dinv2d, b2, g2, be2, W2, False)
    return out


# speedup vs baseline: 18.0336x; 2.0839x over previous
import functools

import jax
import jax.numpy as jnp
from jax import lax
from jax.experimental import pallas as pl
from jax.experimental.pallas import tpu as pltpu
from jax.experimental.pallas import tpu_sc as plsc

NC = 2
NS = 16
NW = NC * NS
EPS = 1e-5
BM = 80
DCH = 800
NPAD = 10240
NR = NPAD // 16
NRC = 128
NCH = NR // NRC


def _rsqrt16(x):
    i = lax.bitcast_convert_type(x, jnp.int32)
    y = lax.bitcast_convert_type(0x5F3759DF - (i >> 1), jnp.float32)
    for _ in range(4):
        y = y * (1.5 - 0.5 * x * y * y)
    return y


def _matmul_body(x_ref, w_ref, o_ref):
    o_ref[...] = jnp.dot(x_ref[...], w_ref[...],
                         preferred_element_type=jnp.float32)


def _matmul(x, w):
    n, _ = x.shape
    m = w.shape[1]
    return pl.pallas_call(
        _matmul_body,
        out_shape=jax.ShapeDtypeStruct((n, m), jnp.float32),
    )(x, w)


def _combine_body(with_mm, p_ref, h_ref, dinv_ref, b_ref, g_ref, be_ref,
                  w_ref, o_ref):
    dinv = dinv_ref[...]
    a = p_ref[0] + p_ref[1] + h_ref[...] * (dinv * dinv) + b_ref[...]
    mean = jnp.mean(a, axis=0, keepdims=True)
    var = jnp.mean((a - mean) * (a - mean), axis=0, keepdims=True)
    xh = (a - mean) * lax.rsqrt(var + EPS)
    y = jnp.maximum(xh * g_ref[...] + be_ref[...], 0.0)
    if with_mm:
        y = jnp.dot(y, w_ref[...], preferred_element_type=jnp.float32)
    o_ref[...] = y


def _combine(p, h, dinv2d, b, g, be, w, with_mm):
    n, d = h.shape
    m = w.shape[1] if with_mm else d
    return pl.pallas_call(
        functools.partial(_combine_body, with_mm),
        out_shape=jax.ShapeDtypeStruct((n, m), jnp.float32),
    )(p, h, dinv2d, b.reshape(1, d), g.reshape(1, d), be.reshape(1, d), w)


def _sc_body(n, e, d, compute_deg,
             meta_h, col_h, ew_h, h_h, dinv_in_h,  znd_h,
             part_h, dinv_out_h,
             meta0, meta1, scidx0, scidx1, ridx0, ridx1, rows0, rows1,
             dinv_v, colv0, colv1, ewv0, ewv1, dacc,
             idxp0, idxp1, idxp2, idxp3, idxp4,
             out_s, deg_s,
             zsem, msem0, msem1, gsem0, gsem1, ssem0, ssem1,
             dsem0, dsem1, psem):
    c = lax.axis_index("c")
    s = lax.axis_index("s")
    wid = c * NS + s
    epw = e // NW
    nb = epw // BM
    b0 = wid * nb
    iot = lax.iota(jnp.int32, 16)

    @pl.when(s == 0)
    def _():
        pltpu.async_copy(znd_h, out_s, zsem)

    if compute_deg:
        zero16 = jnp.zeros((16,), jnp.float32)

        def z1(i, cr):
            dacc[i] = zero16
            return cr

        lax.fori_loop(0, NR, z1, 0)
        zseg = NR // NS
        pltpu.sync_copy(dacc.at[pl.ds(0, zseg)],
                        deg_s.at[pl.ds(s * zseg, zseg)])
        idxps = (idxp0, idxp1, idxp2, idxp3, idxp4)
        for ch in range(NCH):
            for j in range(NRC // 16):
                idxps[ch][pl.ds(j * 16, 16)] = ch * NRC + j * 16 + iot
        plsc.subcore_barrier()

        eps_sc = e // NS
        nch = eps_sc // DCH
        cbs = (colv0, colv1)
        ebs = (ewv0, ewv1)
        dsm = (dsem0, dsem1)
        base0 = s * eps_sc
        pltpu.async_copy(col_h.at[pl.ds(base0, DCH)], colv0, dsem0)
        pltpu.async_copy(ew_h.at[pl.ds(base0, DCH)], ewv0, dsem0)
        for ch in range(nch):
            b = ch % 2
            if ch + 1 < nch:
                nbase = base0 + (ch + 1) * DCH
                pltpu.async_copy(col_h.at[pl.ds(nbase, DCH)],
                                 cbs[1 - b], dsm[1 - b])
                pltpu.async_copy(ew_h.at[pl.ds(nbase, DCH)],
                                 ebs[1 - b], dsm[1 - b])
            pltpu.make_async_copy(col_h.at[pl.ds(0, DCH)], cbs[b],
                                  dsm[b]).wait()
            pltpu.make_async_copy(ew_h.at[pl.ds(0, DCH)], ebs[b],
                                  dsm[b]).wait()

            def dstep(k, cr):
                sl = pl.ds(k * 16, 16)
                cv = cbs[b][sl]
                w = ebs[b][sl]
                hi = cv >> 4
                lo = cv & 15
                for g in range(16):
                    plsc.addupdate_scatter(dacc, [hi, lo], w,
                                           mask=iot == g)
                return cr

            lax.fori_loop(0, DCH // 16, dstep, 0)

        for ch in range(NCH):
            pltpu.async_copy(dacc.at[pl.ds(ch * NRC, NRC)],
                             deg_s.at[idxps[ch]], psem, add=True)
        for ch in range(NCH):
            pltpu.make_async_copy(dacc.at[pl.ds(ch * NRC, NRC)],
                                  deg_s.at[idxps[ch]], psem).wait()
        plsc.subcore_barrier()

        pltpu.sync_copy(deg_s, dinv_v)

        def nred(i, cr):
            dinv_v[i] = _rsqrt16(dinv_v[i] + 1.0)
            return cr

        lax.fori_loop(0, NR, nred, 0)

        @pl.when(jnp.logical_and(c == 0, s == 0))
        def _():
            pltpu.sync_copy(dinv_v, dinv_out_h)
    else:
        pltpu.sync_copy(dinv_in_h, dinv_v)

    @pl.when(s == 0)
    def _():
        pltpu.make_async_copy(znd_h, out_s, zsem).wait()

    plsc.subcore_barrier()

    bufs = ((meta0, scidx0, ridx0, rows0, msem0, gsem0, ssem0),
            (meta1, scidx1, ridx1, rows1, msem1, gsem1, ssem1))

    def fill_ridx(meta, ridx):
        for j in range(BM // 16):
            sl = pl.ds(j * 16, 16)
            ridx[sl] = meta[0, sl]

    pltpu.async_copy(meta_h.at[b0], meta0, msem0)
    pltpu.async_copy(meta_h.at[b0 + 1], meta1, msem1)
    pltpu.make_async_copy(meta_h.at[0], meta0, msem0).wait()
    fill_ridx(meta0, ridx0)
    pltpu.async_copy(h_h.at[ridx0], rows0, gsem0)

    def step(i, cur, nxt):
        meta, scidx, ridx, rows, msem, gsem, ssem = cur
        nmeta, nscidx, nridx, nrows, nmsem, ngsem, nssem = nxt
        pltpu.make_async_copy(h_h.at[ridx], rows, gsem).wait()

        def norm16(j, cr):
            sl = pl.ds(j * 16, 16)
            r = meta[0, sl]
            c2 = meta[1, sl]
            w = lax.bitcast_convert_type(meta[2, sl], jnp.float32)
            scidx[sl] = c2
            dr = plsc.load_gather(dinv_v, [r >> 4, r & 15])
            dc = plsc.load_gather(dinv_v, [c2 >> 4, c2 & 15])
            meta[2, sl] = lax.bitcast_convert_type(dr * w * dc, jnp.int32)
            return cr

        lax.fori_loop(0, BM // 16, norm16, 0)

        def scale16(j, cr):
            nv = lax.bitcast_convert_type(meta[2, pl.ds(j * 16, 16)],
                                          jnp.float32)
            for k in range(16):
                f = nv[k]
                for q in range(d // 16):
                    sl2 = (j * 16 + k, pl.ds(q * 16, 16))
                    rows[sl2] = rows[sl2] * f
            return cr

        lax.fori_loop(0, BM // 16, scale16, 0)
        pltpu.async_copy(rows, out_s.at[scidx], ssem, add=True)

        @pl.when(i + 2 < nb)
        def _():
            pltpu.async_copy(meta_h.at[b0 + i + 2], meta, msem)

        @pl.when(i + 1 < nb)
        def _():
            @pl.when(i > 0)
            def _():
                pltpu.make_async_copy(nrows, out_s.at[nscidx], nssem).wait()

            pltpu.make_async_copy(meta_h.at[0], nmeta, nmsem).wait()
            fill_ridx(nmeta, nridx)
            pltpu.async_copy(h_h.at[nridx], nrows, ngsem)

    def mbody(i, cr):
        @pl.when(i % 2 == 0)
        def _():
            step(i, bufs[0], bufs[1])

        @pl.when(i % 2 == 1)
        def _():
            step(i, bufs[1], bufs[0])

        return cr

    lax.fori_loop(0, nb, mbody, 0)
    pltpu.make_async_copy(rows0, out_s.at[scidx0], ssem0).wait()
    pltpu.make_async_copy(rows1, out_s.at[scidx1], ssem1).wait()
    plsc.subcore_barrier()

    @pl.when(s == 0)
    def _():
        pltpu.sync_copy(out_s, part_h.at[c])


def _sc_pass(meta, col, ew, h, dinv2d, compute_deg):
    n, d = h.shape
    e = col.shape[0]
    mesh = plsc.VectorSubcoreMesh(core_axis_name="c", subcore_axis_name="s")
    znd = jnp.zeros((n, d), jnp.float32)
    if dinv2d is None:
        dinv2d = jnp.zeros((NR, 16), jnp.float32)
    out_type = (jax.ShapeDtypeStruct((NC, n, d), jnp.float32),
                jax.ShapeDtypeStruct((NR, 16), jnp.float32))
    scratch = [
        pltpu.VMEM((3, BM), jnp.int32),
        pltpu.VMEM((3, BM), jnp.int32),
        pltpu.VMEM((BM,), jnp.int32),
        pltpu.VMEM((BM,), jnp.int32),
        pltpu.VMEM((BM,), jnp.int32),
        pltpu.VMEM((BM,), jnp.int32),
        pltpu.VMEM((BM, d), jnp.float32),
        pltpu.VMEM((BM, d), jnp.float32),
        pltpu.VMEM((NR, 16), jnp.float32),
        pltpu.VMEM((DCH,), jnp.int32),
        pltpu.VMEM((DCH,), jnp.int32),
        pltpu.VMEM((DCH,), jnp.float32),
        pltpu.VMEM((DCH,), jnp.float32),
        pltpu.VMEM((NR, 16), jnp.float32),
        pltpu.VMEM((NRC,), jnp.int32),
        pltpu.VMEM((NRC,), jnp.int32),
        pltpu.VMEM((NRC,), jnp.int32),
        pltpu.VMEM((NRC,), jnp.int32),
        pltpu.VMEM((NRC,), jnp.int32),
        pltpu.VMEM_SHARED((n, d), jnp.float32),
        pltpu.VMEM_SHARED((NR, 16), jnp.float32),
        pltpu.SemaphoreType.DMA,
        pltpu.SemaphoreType.DMA,
        pltpu.SemaphoreType.DMA,
        pltpu.SemaphoreType.DMA,
        pltpu.SemaphoreType.DMA,
        pltpu.SemaphoreType.DMA,
        pltpu.SemaphoreType.DMA,
        pltpu.SemaphoreType.DMA,
        pltpu.SemaphoreType.DMA,
        pltpu.SemaphoreType.DMA,
    ]
    fn = pl.kernel(
        functools.partial(_sc_body, n, e, d, compute_deg),
        out_type=out_type,
        mesh=mesh,
        scratch_types=scratch,
        compiler_params=pltpu.CompilerParams(needs_layout_passes=False,
                                             use_tc_tiling_on_sc=False),
    )
    return fn(meta, col, ew, h, dinv2d, znd)


def kernel(x, edge_index, edge_weight, W1, b1, g1, be1, W2, b2, g2, be2):
    row = edge_index[0].astype(jnp.int32)
    col = edge_index[1].astype(jnp.int32)
    ew = edge_weight.astype(jnp.float32)
    e = col.shape[0]
    n = x.shape[0]
    meta = jnp.stack([row.reshape(e // BM, BM), col.reshape(e // BM, BM),
                      lax.bitcast_convert_type(ew, jnp.int32)
                         .reshape(e // BM, BM)], axis=1)

    h1 = _matmul(x, W1)
    p1, dinv2d = _sc_pass(meta, col, ew, h1, None, True)
    dinv_col = dinv2d.reshape(-1)[:n].reshape(-1, 1)
    h2 = _combine(p1, h1, dinv_col, b1, g1, be1, W2, True)
    p2, _ = _sc_pass(meta, col, ew, h2, dinv2d, False)
    out = _combine(p2, h2, dinv_col, b2, g2, be2, W2, False)
    return out


# speedup vs baseline: 25.7214x; 1.4263x over previous
import functools

import jax
import jax.numpy as jnp
from jax import lax
from jax.experimental import pallas as pl
from jax.experimental.pallas import tpu as pltpu
from jax.experimental.pallas import tpu_sc as plsc

NC = 2
NS = 16
NW = NC * NS
EPS = 1e-5
BM = 80
DCH = 800
NPAD = 10240
NR = NPAD // 16
NRC = 128
NCH = NR // NRC


def _rsqrt16(x):
    i = lax.bitcast_convert_type(x, jnp.int32)
    y = lax.bitcast_convert_type(0x5F3759DF - (i >> 1), jnp.float32)
    for _ in range(4):
        y = y * (1.5 - 0.5 * x * y * y)
    return y


def _matmul_body(x_ref, w_ref, o_ref):
    o_ref[...] = jnp.dot(x_ref[...], w_ref[...],
                         preferred_element_type=jnp.float32)


def _matmul(x, w):
    n, _ = x.shape
    m = w.shape[1]
    return pl.pallas_call(
        _matmul_body,
        out_shape=jax.ShapeDtypeStruct((n, m), jnp.float32),
    )(x, w)


def _combine_body(with_mm, p_ref, h_ref, dinv_ref, b_ref, g_ref, be_ref,
                  w_ref, o_ref):
    dinv = dinv_ref[...]
    a = p_ref[0] + p_ref[1] + h_ref[...] * (dinv * dinv) + b_ref[...]
    mean = jnp.mean(a, axis=0, keepdims=True)
    var = jnp.mean((a - mean) * (a - mean), axis=0, keepdims=True)
    xh = (a - mean) * lax.rsqrt(var + EPS)
    y = jnp.maximum(xh * g_ref[...] + be_ref[...], 0.0)
    if with_mm:
        y = jnp.dot(y, w_ref[...], preferred_element_type=jnp.float32)
    o_ref[...] = y


def _combine(p, h, dinv2d, b, g, be, w, with_mm):
    n, d = h.shape
    m = w.shape[1] if with_mm else d
    return pl.pallas_call(
        functools.partial(_combine_body, with_mm),
        out_shape=jax.ShapeDtypeStruct((n, m), jnp.float32),
    )(p, h, dinv2d, b.reshape(1, d), g.reshape(1, d), be.reshape(1, d), w)


def _sc_body(n, e, d, compute_deg,
             meta_h, col_h, ew_h, h_h, dinv_in_h,
             part_h, dinv_out_h,
             meta0, meta1, meta2, scidx0, scidx1, scidx2,
             ridx0, ridx1, ridx2, rows0, rows1, rows2,
             dinv_v, colv0, colv1, ewv0, ewv1, idxp,
             out_s, deg_s,
             msem0, msem1, msem2, gsem0, gsem1, gsem2,
             ssem0, ssem1, ssem2, dsem0, dsem1, psem):
    c = lax.axis_index("c")
    s = lax.axis_index("s")
    wid = c * NS + s
    epw = e // NW
    nb = epw // BM
    b0 = wid * nb
    iot = lax.iota(jnp.int32, 16)
    zero16 = jnp.zeros((16,), jnp.float32)
    zrows = n // NS

    def zero_rows0():
        def z1(i, cr):
            rows0[i // 8, pl.ds((i % 8) * 16, 16)] = zero16
            return cr
        lax.fori_loop(0, BM * d // 16, z1, 0)

    def zero_out_slice():
        nfull = zrows // BM
        for q in range(nfull):
            pltpu.sync_copy(rows0.at[pl.ds(0, BM)],
                            out_s.at[pl.ds(s * zrows + q * BM, BM)])
        rem = zrows - nfull * BM
        if rem:
            pltpu.sync_copy(rows0.at[pl.ds(0, rem)],
                            out_s.at[pl.ds(s * zrows + nfull * BM, rem)])

    if compute_deg:
        zero_rows0()
        zseg = (NPAD // d) // NS
        pltpu.sync_copy(rows0.at[pl.ds(0, zseg)],
                        deg_s.at[pl.ds(s * zseg, zseg)])
        for j in range((NPAD // d) // 16):
            idxp[pl.ds(j * 16, 16)] = j * 16 + iot
        plsc.subcore_barrier()

        eps_sc = e // NS
        nch = eps_sc // DCH
        cbs = (colv0, colv1)
        ebs = (ewv0, ewv1)
        dsm = (dsem0, dsem1)
        base0 = s * eps_sc
        pltpu.async_copy(col_h.at[pl.ds(base0, DCH)], colv0, dsem0)
        pltpu.async_copy(ew_h.at[pl.ds(base0, DCH)], ewv0, dsem0)
        for ch in range(nch):
            b = ch % 2
            if ch + 1 < nch:
                nbase = base0 + (ch + 1) * DCH
                pltpu.async_copy(col_h.at[pl.ds(nbase, DCH)],
                                 cbs[1 - b], dsm[1 - b])
                pltpu.async_copy(ew_h.at[pl.ds(nbase, DCH)],
                                 ebs[1 - b], dsm[1 - b])
            pltpu.make_async_copy(col_h.at[pl.ds(0, DCH)], cbs[b],
                                  dsm[b]).wait()
            pltpu.make_async_copy(ew_h.at[pl.ds(0, DCH)], ebs[b],
                                  dsm[b]).wait()

            def dstep(k, cr):
                sl = pl.ds(k * 16, 16)
                cv = cbs[b][sl]
                w = ebs[b][sl]
                hi = cv >> 7
                lo = cv & (d - 1)
                for g in range(16):
                    plsc.addupdate_scatter(rows0, [hi, lo], w,
                                           mask=iot == g)
                return cr

            lax.fori_loop(0, DCH // 16, dstep, 0)

        pltpu.async_copy(rows0, deg_s.at[idxp], psem, add=True)
        pltpu.make_async_copy(rows0, deg_s.at[idxp], psem).wait()
        plsc.subcore_barrier()

        pltpu.sync_copy(deg_s, dinv_v)

        def nred(i, cr):
            sl = (i // 8, pl.ds((i % 8) * 16, 16))
            dinv_v[sl] = _rsqrt16(dinv_v[sl] + 1.0)
            return cr

        lax.fori_loop(0, BM * d // 16, nred, 0)

        @pl.when(jnp.logical_and(c == 0, s == 0))
        def _():
            pltpu.sync_copy(dinv_v, dinv_out_h)

        zero_rows0()
        zero_out_slice()
    else:
        pltpu.sync_copy(dinv_in_h, dinv_v)
        zero_rows0()
        zero_out_slice()

    plsc.subcore_barrier()

    bufs = ((meta0, scidx0, ridx0, rows0, msem0, gsem0, ssem0),
            (meta1, scidx1, ridx1, rows1, msem1, gsem1, ssem1),
            (meta2, scidx2, ridx2, rows2, msem2, gsem2, ssem2))

    def fill_ridx(meta, ridx):
        for j in range(BM // 16):
            sl = pl.ds(j * 16, 16)
            ridx[sl] = meta[0, sl]

    for j in range(3):
        pltpu.async_copy(meta_h.at[b0 + j], bufs[j][0], bufs[j][4])
    for j in range(2):
        meta, scidx, ridx, rows, msem, gsem, ssem = bufs[j]
        pltpu.make_async_copy(meta_h.at[0], meta, msem).wait()
        fill_ridx(meta, ridx)
        pltpu.async_copy(h_h.at[ridx], rows, gsem)

    def step(i, cur, nx2):
        meta, scidx, ridx, rows, msem, gsem, ssem = cur
        nmeta, nscidx, nridx, nrows, nmsem, ngsem, nssem = nx2
        pltpu.make_async_copy(h_h.at[ridx], rows, gsem).wait()

        def norm16(j, cr):
            sl = pl.ds(j * 16, 16)
            r = meta[0, sl]
            c2 = meta[1, sl]
            w = lax.bitcast_convert_type(meta[2, sl], jnp.float32)
            scidx[sl] = c2
            dr = plsc.load_gather(dinv_v, [r >> 7, r & (d - 1)])
            dc = plsc.load_gather(dinv_v, [c2 >> 7, c2 & (d - 1)])
            meta[2, sl] = lax.bitcast_convert_type(dr * w * dc, jnp.int32)
            return cr

        lax.fori_loop(0, BM // 16, norm16, 0)

        def scale16(j, cr):
            nv = lax.bitcast_convert_type(meta[2, pl.ds(j * 16, 16)],
                                          jnp.float32)
            for k in range(16):
                f = nv[k]
                for q in range(d // 16):
                    sl2 = (j * 16 + k, pl.ds(q * 16, 16))
                    rows[sl2] = rows[sl2] * f
            return cr

        lax.fori_loop(0, BM // 16, scale16, 0)
        pltpu.async_copy(rows, out_s.at[scidx], ssem, add=True)

        @pl.when(i + 3 < nb)
        def _():
            pltpu.async_copy(meta_h.at[b0 + i + 3], meta, msem)

        @pl.when(i + 2 < nb)
        def _():
            @pl.when(i > 0)
            def _():
                pltpu.make_async_copy(nrows, out_s.at[nscidx], nssem).wait()

            pltpu.make_async_copy(meta_h.at[0], nmeta, nmsem).wait()
            fill_ridx(nmeta, nridx)
            pltpu.async_copy(h_h.at[nridx], nrows, ngsem)

    def mbody(i, cr):
        for k in range(3):
            @pl.when(i % 3 == k)
            def _():
                step(i, bufs[k], bufs[(k + 2) % 3])

        return cr

    lax.fori_loop(0, nb, mbody, 0)
    for j in ((nb - 2) % 3, (nb - 1) % 3):
        meta, scidx, ridx, rows, msem, gsem, ssem = bufs[j]
        pltpu.make_async_copy(rows, out_s.at[scidx], ssem).wait()
    plsc.subcore_barrier()

    @pl.when(s == 0)
    def _():
        pltpu.sync_copy(out_s, part_h.at[c])


def _sc_pass(meta, col, ew, h, dinv2d, compute_deg):
    n, d = h.shape
    e = col.shape[0]
    mesh = plsc.VectorSubcoreMesh(core_axis_name="c", subcore_axis_name="s")
    if dinv2d is None:
        dinv2d = jnp.zeros((NPAD // d, d), jnp.float32)
    out_type = (jax.ShapeDtypeStruct((NC, n, d), jnp.float32),
                jax.ShapeDtypeStruct((NPAD // d, d), jnp.float32))
    scratch = [
        pltpu.VMEM((3, BM), jnp.int32),
        pltpu.VMEM((3, BM), jnp.int32),
        pltpu.VMEM((3, BM), jnp.int32),
        pltpu.VMEM((BM,), jnp.int32),
        pltpu.VMEM((BM,), jnp.int32),
        pltpu.VMEM((BM,), jnp.int32),
        pltpu.VMEM((BM,), jnp.int32),
        pltpu.VMEM((BM,), jnp.int32),
        pltpu.VMEM((BM,), jnp.int32),
        pltpu.VMEM((BM, d), jnp.float32),
        pltpu.VMEM((BM, d), jnp.float32),
        pltpu.VMEM((BM, d), jnp.float32),
        pltpu.VMEM((NPAD // d, d), jnp.float32),
        pltpu.VMEM((DCH,), jnp.int32),
        pltpu.VMEM((DCH,), jnp.int32),
        pltpu.VMEM((DCH,), jnp.float32),
        pltpu.VMEM((DCH,), jnp.float32),
        pltpu.VMEM((NPAD // d,), jnp.int32),
        pltpu.VMEM_SHARED((n, d), jnp.float32),
        pltpu.VMEM_SHARED((NPAD // d, d), jnp.float32),
        pltpu.SemaphoreType.DMA,
        pltpu.SemaphoreType.DMA,
        pltpu.SemaphoreType.DMA,
        pltpu.SemaphoreType.DMA,
        pltpu.SemaphoreType.DMA,
        pltpu.SemaphoreType.DMA,
        pltpu.SemaphoreType.DMA,
        pltpu.SemaphoreType.DMA,
        pltpu.SemaphoreType.DMA,
        pltpu.SemaphoreType.DMA,
        pltpu.SemaphoreType.DMA,
        pltpu.SemaphoreType.DMA,
    ]
    fn = pl.kernel(
        functools.partial(_sc_body, n, e, d, compute_deg),
        out_type=out_type,
        mesh=mesh,
        scratch_types=scratch,
        compiler_params=pltpu.CompilerParams(needs_layout_passes=False,
                                             use_tc_tiling_on_sc=False),
    )
    return fn(meta, col, ew, h, dinv2d)


def kernel(x, edge_index, edge_weight, W1, b1, g1, be1, W2, b2, g2, be2):
    row = edge_index[0].astype(jnp.int32)
    col = edge_index[1].astype(jnp.int32)
    ew = edge_weight.astype(jnp.float32)
    e = col.shape[0]
    n = x.shape[0]
    meta = jnp.stack([row.reshape(e // BM, BM), col.reshape(e // BM, BM),
                      lax.bitcast_convert_type(ew, jnp.int32)
                         .reshape(e // BM, BM)], axis=1)

    h1 = _matmul(x, W1)
    p1, dinv2d = _sc_pass(meta, col, ew, h1, None, True)
    dinv_col = dinv2d.reshape(-1)[:n].reshape(-1, 1)
    h2 = _combine(p1, h1, dinv_col, b1, g1, be1, W2, True)
    p2, _ = _sc_pass(meta, col, ew, h2, dinv2d, False)
    out = _combine(p2, h2, dinv_col, b2, g2, be2, W2, False)
    return out


# speedup vs baseline: 27.6499x; 1.0750x over previous
import functools

import jax
import jax.numpy as jnp
from jax import lax
from jax.experimental import pallas as pl
from jax.experimental.pallas import tpu as pltpu
from jax.experimental.pallas import tpu_sc as plsc

NC = 2
NS = 16
NW = NC * NS
EPS = 1e-5
BM = 80
DCH = 2000
NPAD = 10240
NB_ROWS = 4


def _mmprep_body(x_ref, w_ref, dg_ref, hp_ref, dinv_ref):
    dinv = lax.rsqrt(dg_ref[0] + dg_ref[1] + 1.0)
    h = jnp.dot(x_ref[...], w_ref[...], preferred_element_type=jnp.float32)
    hp_ref[...] = h * dinv
    dinv_ref[...] = dinv


def _mmprep(x, w, deg2):
    n, _ = x.shape
    m = w.shape[1]
    return pl.pallas_call(
        _mmprep_body,
        out_shape=(jax.ShapeDtypeStruct((n, m), jnp.float32),
                   jax.ShapeDtypeStruct((n, 1), jnp.float32)),
    )(x, w, deg2)


def _combine_body(with_mm, p_ref, hp_ref, dinv_ref, b_ref, g_ref, be_ref,
                  w_ref, o_ref):
    dinv = dinv_ref[...]
    a = (p_ref[0] + p_ref[1] + hp_ref[...]) * dinv + b_ref[...]
    mean = jnp.mean(a, axis=0, keepdims=True)
    var = jnp.mean((a - mean) * (a - mean), axis=0, keepdims=True)
    xh = (a - mean) * lax.rsqrt(var + EPS)
    y = jnp.maximum(xh * g_ref[...] + be_ref[...], 0.0)
    if with_mm:
        y = jnp.dot(y, w_ref[...],
                    preferred_element_type=jnp.float32) * dinv
    o_ref[...] = y


def _combine(p, hp, dinv_col, b, g, be, w, with_mm):
    n, d = hp.shape
    m = w.shape[1] if with_mm else d
    return pl.pallas_call(
        functools.partial(_combine_body, with_mm),
        out_shape=jax.ShapeDtypeStruct((n, m), jnp.float32),
    )(p, hp, dinv_col, b.reshape(1, d), g.reshape(1, d), be.reshape(1, d), w)


def _deg_body(e, d, col_h, ew_h, deg_h,
              hist, colv0, colv1, ewv0, ewv1, idxp,
              deg_s, dsem0, dsem1, psem):
    c = lax.axis_index("c")
    s = lax.axis_index("s")
    wid = c * NS + s
    epw = e // NW
    iot = lax.iota(jnp.int32, 16)
    zero16 = jnp.zeros((16,), jnp.float32)
    ndr = NPAD // d

    def z1(i, cr):
        hist[i // 8, pl.ds((i % 8) * 16, 16)] = zero16
        return cr

    lax.fori_loop(0, NPAD // 16, z1, 0)
    zseg = ndr // NS
    pltpu.sync_copy(hist.at[pl.ds(0, zseg)],
                    deg_s.at[pl.ds(s * zseg, zseg)])
    for j in range(ndr // 16):
        idxp[pl.ds(j * 16, 16)] = j * 16 + iot
    plsc.subcore_barrier()

    nch = epw // DCH
    cbs = (colv0, colv1)
    ebs = (ewv0, ewv1)
    dsm = (dsem0, dsem1)
    base0 = wid * epw
    pltpu.async_copy(col_h.at[pl.ds(base0, DCH)], colv0, dsem0)
    pltpu.async_copy(ew_h.at[pl.ds(base0, DCH)], ewv0, dsem0)
    for ch in range(nch):
        b = ch % 2
        if ch + 1 < nch:
            nbase = base0 + (ch + 1) * DCH
            pltpu.async_copy(col_h.at[pl.ds(nbase, DCH)],
                             cbs[1 - b], dsm[1 - b])
            pltpu.async_copy(ew_h.at[pl.ds(nbase, DCH)],
                             ebs[1 - b], dsm[1 - b])
        pltpu.make_async_copy(col_h.at[pl.ds(0, DCH)], cbs[b],
                              dsm[b]).wait()
        pltpu.make_async_copy(ew_h.at[pl.ds(0, DCH)], ebs[b],
                              dsm[b]).wait()

        def dstep(k, cr):
            sl = pl.ds(k * 16, 16)
            cv = cbs[b][sl]
            w = ebs[b][sl]
            hi = cv >> 7
            lo = cv & (d - 1)
            for g in range(16):
                plsc.addupdate_scatter(hist, [hi, lo], w, mask=iot == g)
            return cr

        lax.fori_loop(0, DCH // 16, dstep, 0)

    pltpu.async_copy(hist, deg_s.at[idxp], psem, add=True)
    pltpu.make_async_copy(hist, deg_s.at[idxp], psem).wait()
    plsc.subcore_barrier()

    @pl.when(s == 0)
    def _():
        pltpu.sync_copy(deg_s, deg_h.at[c])


def _sc_deg(col, ew, d):
    e = col.shape[0]
    ndr = NPAD // d
    mesh = plsc.VectorSubcoreMesh(core_axis_name="c", subcore_axis_name="s")
    fn = pl.kernel(
        functools.partial(_deg_body, e, d),
        out_type=jax.ShapeDtypeStruct((NC, ndr, d), jnp.float32),
        mesh=mesh,
        scratch_types=[
            pltpu.VMEM((ndr, d), jnp.float32),
            pltpu.VMEM((DCH,), jnp.int32),
            pltpu.VMEM((DCH,), jnp.int32),
            pltpu.VMEM((DCH,), jnp.float32),
            pltpu.VMEM((DCH,), jnp.float32),
            pltpu.VMEM((ndr,), jnp.int32),
            pltpu.VMEM_SHARED((ndr, d), jnp.float32),
            pltpu.SemaphoreType.DMA,
            pltpu.SemaphoreType.DMA,
            pltpu.SemaphoreType.DMA,
        ],
        compiler_params=pltpu.CompilerParams(needs_layout_passes=False,
                                             use_tc_tiling_on_sc=False),
    )
    return fn(col, ew)


def _msg_body(n, e, d,
              meta_h, h_h, part_h,
              meta0, meta1, meta2, meta3,
              scidx0, scidx1, scidx2, scidx3,
              ridx0, ridx1, ridx2, ridx3,
              rows0, rows1, rows2, rows3,
              out_s,
              msem0, msem1, msem2, msem3,
              gsem0, gsem1, gsem2, gsem3,
              ssem0, ssem1, ssem2, ssem3):
    c = lax.axis_index("c")
    s = lax.axis_index("s")
    wid = c * NS + s
    epw = e // NW
    nb = epw // BM
    b0 = wid * nb
    zero16 = jnp.zeros((16,), jnp.float32)
    zrows = n // NS

    def z1(i, cr):
        rows0[i // 8, pl.ds((i % 8) * 16, 16)] = zero16
        return cr

    lax.fori_loop(0, BM * d // 16, z1, 0)
    nfull = zrows // BM
    for q in range(nfull):
        pltpu.sync_copy(rows0.at[pl.ds(0, BM)],
                        out_s.at[pl.ds(s * zrows + q * BM, BM)])
    rem = zrows - nfull * BM
    if rem:
        pltpu.sync_copy(rows0.at[pl.ds(0, rem)],
                        out_s.at[pl.ds(s * zrows + nfull * BM, rem)])
    plsc.subcore_barrier()

    bufs = ((meta0, scidx0, ridx0, rows0, msem0, gsem0, ssem0),
            (meta1, scidx1, ridx1, rows1, msem1, gsem1, ssem1),
            (meta2, scidx2, ridx2, rows2, msem2, gsem2, ssem2),
            (meta3, scidx3, ridx3, rows3, msem3, gsem3, ssem3))

    def fill(meta, off, dst):
        for j in range(BM // 16):
            sl = pl.ds(j * 16, 16)
            dst[sl] = meta[pl.ds(off + j * 16, 16)]

    for j in range(NB_ROWS):
        pltpu.async_copy(meta_h.at[pl.ds((b0 + j) * 3 * BM, 3 * BM)],
                         bufs[j][0], bufs[j][4])
    for j in range(2):
        meta, scidx, ridx, rows, msem, gsem, ssem = bufs[j]
        pltpu.make_async_copy(meta_h.at[pl.ds(0, 3 * BM)], meta,
                              msem).wait()
        fill(meta, 0, ridx)
        pltpu.async_copy(h_h.at[ridx], rows, gsem)

    def step(i, cur, nx2):
        meta, scidx, ridx, rows, msem, gsem, ssem = cur
        nmeta, nscidx, nridx, nrows, nmsem, ngsem, nssem = nx2
        pltpu.make_async_copy(h_h.at[ridx], rows, gsem).wait()
        fill(meta, BM, scidx)

        def scale16(j, cr):
            ev = lax.bitcast_convert_type(
                meta[pl.ds(2 * BM + j * 16, 16)], jnp.float32)
            for k in range(16):
                f = ev[k]
                for q in range(d // 16):
                    sl2 = (j * 16 + k, pl.ds(q * 16, 16))
                    rows[sl2] = rows[sl2] * f
            return cr

        lax.fori_loop(0, BM // 16, scale16, 0)
        pltpu.async_copy(rows, out_s.at[scidx], ssem, add=True)

        @pl.when(i + NB_ROWS < nb)
        def _():
            pltpu.async_copy(
                meta_h.at[pl.ds((b0 + i + NB_ROWS) * 3 * BM, 3 * BM)],
                meta, msem)

        @pl.when(i + 2 < nb)
        def _():
            @pl.when(i > 1)
            def _():
                pltpu.make_async_copy(nrows, out_s.at[nscidx], nssem).wait()

            pltpu.make_async_copy(meta_h.at[pl.ds(0, 3 * BM)], nmeta,
                                  nmsem).wait()
            fill(nmeta, 0, nridx)
            pltpu.async_copy(h_h.at[nridx], nrows, ngsem)

    def mbody(i, cr):
        for k in range(NB_ROWS):
            @pl.when(i % NB_ROWS == k)
            def _():
                step(i, bufs[k], bufs[(k + 2) % NB_ROWS])

        return cr

    lax.fori_loop(0, nb, mbody, 0)
    for j in ((nb - 2) % NB_ROWS, (nb - 1) % NB_ROWS):
        meta, scidx, ridx, rows, msem, gsem, ssem = bufs[j]
        pltpu.make_async_copy(rows, out_s.at[scidx], ssem).wait()
    plsc.subcore_barrier()

    @pl.when(s == 0)
    def _():
        pltpu.sync_copy(out_s, part_h.at[c])


def _sc_msg(meta, hp):
    n, d = hp.shape
    e = meta.shape[0] // 3
    mesh = plsc.VectorSubcoreMesh(core_axis_name="c", subcore_axis_name="s")
    scratch = (
        [pltpu.VMEM((3 * BM,), jnp.int32) for _ in range(NB_ROWS)]
        + [pltpu.VMEM((BM,), jnp.int32) for _ in range(NB_ROWS)]
        + [pltpu.VMEM((BM,), jnp.int32) for _ in range(NB_ROWS)]
        + [pltpu.VMEM((BM, d), jnp.float32) for _ in range(NB_ROWS)]
        + [pltpu.VMEM_SHARED((n, d), jnp.float32)]
        + [pltpu.SemaphoreType.DMA for _ in range(3 * NB_ROWS)]
    )
    fn = pl.kernel(
        functools.partial(_msg_body, n, e, d),
        out_type=jax.ShapeDtypeStruct((NC, n, d), jnp.float32),
        mesh=mesh,
        scratch_types=scratch,
        compiler_params=pltpu.CompilerParams(needs_layout_passes=False,
                                             use_tc_tiling_on_sc=False),
    )
    return fn(meta, hp)


def kernel(x, edge_index, edge_weight, W1, b1, g1, be1, W2, b2, g2, be2):
    row = edge_index[0].astype(jnp.int32)
    col = edge_index[1].astype(jnp.int32)
    ew = edge_weight.astype(jnp.float32)
    e = col.shape[0]
    n, d = x.shape[0], W1.shape[1]
    meta = jnp.stack([row.reshape(e // BM, BM), col.reshape(e // BM, BM),
                      lax.bitcast_convert_type(ew, jnp.int32)
                         .reshape(e // BM, BM)], axis=1).reshape(-1)

    deg = _sc_deg(col, ew, d)
    deg2 = deg.reshape(NC, -1)[:, :n].reshape(NC, n, 1)
    h1p, dinv_col = _mmprep(x, W1, deg2)
    p1 = _sc_msg(meta, h1p)
    h2p = _combine(p1, h1p, dinv_col, b1, g1, be1, W2, True)
    p2 = _sc_msg(meta, h2p)
    out = _combine(p2, h2p, dinv_col, b2, g2, be2, W2, False)
    return out
